# Initial kernel scaffold; baseline (speedup 1.0000x reference)
import jax
import jax.numpy as jnp
from jax.experimental import pallas as pl


def kernel(input, attention_mask, question_mask, number_mask):
    raise NotImplementedError("write your pallas kernel here")


# speedup vs baseline: 1.6958x; 1.6958x over previous
import functools

import jax
import jax.numpy as jnp
from jax import lax
from jax.experimental import pallas as pl
from jax.experimental.pallas import tpu as pltpu
from jax.experimental.pallas import tpu_sc as plsc

B, S, H, MAXN = 16, 4096, 1024, 64

_NC, _NS, _L = 2, 16, 16
_NW = _NC * _NS
ROWS = 2 * B * MAXN
RPW = ROWS // _NW


def _index_kernel(nm_ref, first_ref, last_ref, scale_ref):
    nm = nm_ref[0]
    ids = lax.broadcasted_iota(jnp.int32, (MAXN, 1), 0) + 1
    match = nm == ids
    pos = lax.broadcasted_iota(jnp.int32, (MAXN, S), 1)
    first = jnp.min(jnp.where(match, pos, S), axis=1)
    last = jnp.max(jnp.where(match, pos, -1), axis=1)
    present = last >= 0
    base = pl.program_id(0) * S
    first_ref[0, 0, :] = jnp.where(present, first, 0) + base
    last_ref[0, 0, :] = jnp.where(present, last, 0) + base
    scale_ref[0, 0, :] = present.astype(jnp.float32)


def _compute_indices(nm3):
    return pl.pallas_call(
        _index_kernel,
        grid=(B,),
        in_specs=[pl.BlockSpec((1, 1, S), lambda b: (b, 0, 0))],
        out_specs=[
            pl.BlockSpec((1, 1, MAXN), lambda b: (b, 0, 0)),
            pl.BlockSpec((1, 1, MAXN), lambda b: (b, 0, 0)),
            pl.BlockSpec((1, 1, MAXN), lambda b: (b, 0, 0)),
        ],
        out_shape=[
            jax.ShapeDtypeStruct((B, 1, MAXN), jnp.int32),
            jax.ShapeDtypeStruct((B, 1, MAXN), jnp.int32),
            jax.ShapeDtypeStruct((B, 1, MAXN), jnp.float32),
        ],
    )(nm3)


def _gather_body(table_hbm, idx_hbm, scale_hbm, out_hbm, idx_v, scale_v,
                 rows_v, sem):
    wid = lax.axis_index("s") * _NC + lax.axis_index("c")
    base = wid * RPW
    pltpu.sync_copy(idx_hbm.at[pl.ds(base, RPW)], idx_v)
    pltpu.sync_copy(scale_hbm.at[pl.ds(base, RPW)], scale_v)
    pltpu.async_copy(table_hbm.at[idx_v], rows_v, sem).wait()

    m = scale_v[pl.ds(0, _L)]
    for g in range(1, RPW // _L):
        m = jnp.minimum(m, scale_v[pl.ds(g * _L, _L)])
    all_present = jnp.min(m)

    @pl.when(all_present < 0.5)
    def _mask_rows():
        def col_body(c, carry):
            off = c * _L
            for r in range(RPW):
                srow = plsc.load_gather(
                    scale_v, [jnp.full((_L,), r, jnp.int32)])
                rows_v[r, pl.ds(off, _L)] = rows_v[r, pl.ds(off, _L)] * srow
            return carry
        lax.fori_loop(0, H // _L, col_body, 0)

    pltpu.sync_copy(rows_v, out_hbm.at[pl.ds(base, RPW)])


@functools.cache
def _gather_rows():
    return pl.kernel(
        _gather_body,
        out_type=jax.ShapeDtypeStruct((ROWS, H), jnp.float32),
        mesh=plsc.VectorSubcoreMesh(core_axis_name="c", subcore_axis_name="s"),
        compiler_params=pltpu.CompilerParams(needs_layout_passes=False),
        scratch_types=[
            pltpu.VMEM((RPW,), jnp.int32),
            pltpu.VMEM((RPW,), jnp.float32),
            pltpu.VMEM((RPW, H), jnp.float32),
            pltpu.SemaphoreType.DMA,
        ],
    )


def kernel(input, attention_mask, question_mask, number_mask):
    nm3 = number_mask.astype(jnp.int32).reshape(B, 1, S)
    first, last, scale = _compute_indices(nm3)
    first = first.reshape(B, MAXN)
    last = last.reshape(B, MAXN)
    idx = jnp.stack([first, last], axis=-1).reshape(ROWS)
    scale2 = jnp.repeat(scale.reshape(B * MAXN), 2)
    table = input.reshape(B * S, H)
    gathered = _gather_rows()(table, idx, scale2)
    return gathered.reshape(B, MAXN, 2 * H)
